# baseline (speedup 1.0000x reference)
import jax
import jax.numpy as jnp
from jax.experimental import pallas as pl
from jax.experimental.pallas import tpu as pltpu

_MIN_BIN = 3.25
_MAX_BIN = 20.75
_NO_BINS = 15
_INF = 100000000.0
_CZ = 128
_N = 1024
_BI = 32
_CH = 16
_K = 64


def _embed_body(xi_ref, xjt_ref, thr_ref, w_ref, o_ref):
    xjt = xjt_ref[...]
    thr = thr_ref[...][0]
    w = w_ref[...]

    for h in range(_BI // _CH):
        xi = xi_ref[h * _CH:(h + 1) * _CH, :]
        d = None
        for c in range(3):
            df = xi[:, c:c + 1] - xjt[c:c + 1, :]
            sq = df * df
            d = sq if d is None else d + sq

        steps = (d[:, :, None] > thr)
        oh = steps.astype(jnp.float32).astype(jnp.bfloat16)
        oh2 = oh.reshape(_CH * _N, _K)
        z = jnp.dot(oh2, w, preferred_element_type=jnp.float32)
        o_ref[h * _CH * _N:(h + 1) * _CH * _N, :] = z


def kernel(x, W, b):
    x2 = x[0]
    xjt = x2.T

    wt = W.T
    wh = wt.astype(jnp.bfloat16)
    wl = (wt - wh.astype(jnp.float32)).astype(jnp.bfloat16)
    b1 = b.reshape(1, _CZ)
    bh = b1.astype(jnp.bfloat16)
    bl = (b1 - bh.astype(jnp.float32)).astype(jnp.bfloat16)
    zeros2 = jnp.zeros((2, _CZ), jnp.bfloat16)
    w64 = jnp.concatenate([wh, wl, -wh, -wl, bh, bl, zeros2], axis=0)

    bins = jnp.linspace(_MIN_BIN, _MAX_BIN, _NO_BINS, dtype=x.dtype)
    sqb = bins ** 2
    up = jnp.concatenate([sqb[1:], jnp.full((1,), _INF, x.dtype)])
    up_pred = jax.lax.bitcast_convert_type(
        jax.lax.bitcast_convert_type(up, jnp.int32) - 1, jnp.float32)
    always = jnp.full((2,), -1.0, x.dtype)
    never = jnp.full((2,), 3.4e38, x.dtype)
    thr64 = jnp.concatenate(
        [sqb, sqb, up_pred, up_pred, always, never]).reshape(1, _K)

    out = pl.pallas_call(
        _embed_body,
        out_shape=jax.ShapeDtypeStruct((_N * _N, _CZ), jnp.float32),
        grid=(_N // _BI,),
        in_specs=[
            pl.BlockSpec((_BI, 3), lambda i: (i, 0)),
            pl.BlockSpec((3, _N), lambda i: (0, 0)),
            pl.BlockSpec((1, _K), lambda i: (0, 0)),
            pl.BlockSpec((_K, _CZ), lambda i: (0, 0)),
        ],
        out_specs=pl.BlockSpec((_BI * _N, _CZ), lambda i: (i, 0)),
        compiler_params=pltpu.CompilerParams(
            dimension_semantics=("arbitrary",),
            vmem_limit_bytes=64 * 1024 * 1024,
        ),
        name="ca_embed",
    )(x2, xjt, thr64, w64)
    return out.reshape(1, _N, _N, _CZ)
